# baseline (speedup 1.0000x reference)
import functools
import jax
import jax.numpy as jnp
from jax import lax
from jax.experimental import pallas as pl
from jax.experimental.pallas import tpu as pltpu
from jax.experimental.pallas import tpu_sc as plsc

N = 1600000
S = 4096
D = 3
EPS = 1e-5

NC = 2
NS = 16
NW = NC * NS
BLK_ROWS = 2048
TOT_BLKS = -(-N // BLK_ROWS)
LAST_BLK = TOT_BLKS - 1
LAST_ROWS = N - LAST_BLK * BLK_ROWS

@functools.cache
def _build_phase1():
    mesh = plsc.VectorSubcoreMesh(
        core_axis_name="c", subcore_axis_name="s", num_cores=NC, num_subcores=NS
    )
    return functools.partial(
        pl.kernel,
        mesh=mesh,
        compiler_params=pltpu.CompilerParams(needs_layout_passes=False),
        out_type=[
            jax.ShapeDtypeStruct((NW, D * S), jnp.float32),
            jax.ShapeDtypeStruct((NW, S), jnp.int32),
        ],
        scratch_types=[
            pltpu.VMEM((D, BLK_ROWS), jnp.float32),
            pltpu.VMEM((BLK_ROWS,), jnp.int32),
            pltpu.VMEM((D, BLK_ROWS), jnp.float32),
            pltpu.VMEM((BLK_ROWS,), jnp.int32),
            pltpu.VMEM((D * S,), jnp.float32),
            pltpu.VMEM((S,), jnp.int32),
            pltpu.VMEM((D * 16,), jnp.float32),
            pltpu.VMEM((16,), jnp.int32),
            pltpu.SMEM((1,), jnp.int32),
            pltpu.SemaphoreType.DMA,
            pltpu.SemaphoreType.DMA,
        ],
    )(_sc_phase1_body)


MAIN_BLKS = 24
EXTRA_W = TOT_BLKS - NW * MAIN_BLKS


def _sc_phase1_body(
    x_hbm, b_hbm, outx_hbm, outc_hbm, xv0, bv0, xv1, bv1, accx, accc,
    carf, cari, opensm, sem0, sem1
):
    wid = lax.axis_index("s") * NC + lax.axis_index("c")

    zf = jnp.zeros((16,), jnp.float32)
    zi = jnp.zeros((16,), jnp.int32)

    def zero_f(i, c):
        accx[pl.ds(i * 16, 16)] = zf
        return c

    def zero_i(i, c):
        accc[pl.ds(i * 16, 16)] = zi
        return c

    lax.fori_loop(0, (D * S) // 16, zero_f, 0)
    lax.fori_loop(0, S // 16, zero_i, 0)
    for m in range(D):
        carf[pl.ds(m * 16, 16)] = jnp.zeros((16,), jnp.float32)
    cari[...] = jnp.zeros((16,), jnp.int32)

    ii = lax.iota(jnp.int32, 16)
    lane0 = ii == 0
    lane15 = ii == 15
    ip1 = jnp.minimum(ii + 1, 15)
    im1 = jnp.maximum(ii - 1, 0)
    ones_i = jnp.ones((16,), jnp.int32)
    zf16 = jnp.zeros((16,), jnp.float32)
    zi16 = jnp.zeros((16,), jnp.int32)

    def start_blk(bufx, bufb, sem, bid):
        off = bid * BLK_ROWS
        pltpu.make_async_copy(x_hbm.at[:, pl.ds(off, BLK_ROWS)], bufx, sem).start()
        pltpu.make_async_copy(b_hbm.at[pl.ds(off, BLK_ROWS)], bufb, sem).start()

    def wait_blk(bufx, bufb, sem):
        pltpu.make_async_copy(x_hbm.at[:, pl.ds(0, BLK_ROWS)], bufx, sem).wait()
        pltpu.make_async_copy(b_hbm.at[pl.ds(0, BLK_ROWS)], bufb, sem).wait()

    def flush():
        openv = jnp.full((16,), opensm[0], jnp.int32)
        plsc.addupdate_scatter(accc, [openv], cari[...])
        cari[...] = zi16
        for m in range(D):
            plsc.addupdate_scatter(accx, [openv + m * S], carf[pl.ds(m * 16, 16)])
            carf[pl.ds(m * 16, 16)] = zf16

    def grp_slow(bt, xs):
        flush()
        nxt = jnp.take_along_axis(bt, ip1, axis=0)
        prv = jnp.take_along_axis(bt, im1, axis=0)
        boundary = (bt != nxt) | lane15
        startm = (bt != prv) | lane0
        s = plsc.cummax(jnp.where(startm, ii, 0))
        plsc.addupdate_scatter(accc, [bt], ii - s + 1, mask=boundary)
        for m in range(D):
            cs = plsc.cumsum(xs[m])
            ecs = cs - xs[m]
            pcs = jnp.take_along_axis(ecs, s, axis=0)
            plsc.addupdate_scatter(accx, [bt + m * S], cs - pcs, mask=boundary)
        opensm[0] = bt[15]

    UNROLL = 4

    def make_grpu(bufx, bufb):
        def grp16(g16):
            xs = [bufx[m, pl.ds(g16, 16)] for m in range(D)]
            bt = bufb[pl.ds(g16, 16)]
            same = (bt[0] == bt[15]) & (bt[0] == opensm[0])

            @pl.when(same)
            def _fast():
                plsc.addupdate(cari.at[pl.ds(0, 16)], ones_i)
                for m in range(D):
                    plsc.addupdate(carf.at[pl.ds(m * 16, 16)], xs[m])

            @pl.when(jnp.logical_not(same))
            def _slow():
                grp_slow(bt, xs)

        def grpu(g, c):
            base = g * (16 * UNROLL)
            bfirst = bufb[pl.ds(base, 16)]
            blast = bufb[pl.ds(base + 16 * (UNROLL - 1), 16)]
            span_same = (bfirst[0] == blast[15]) & (bfirst[0] == opensm[0])

            @pl.when(span_same)
            def _fastspan():
                plsc.addupdate(cari.at[pl.ds(0, 16)], UNROLL * ones_i)
                for m in range(D):
                    acc = bufx[m, pl.ds(base, 16)]
                    for u in range(1, UNROLL):
                        acc = acc + bufx[m, pl.ds(base + u * 16, 16)]
                    plsc.addupdate(carf.at[pl.ds(m * 16, 16)], acc)

            @pl.when(jnp.logical_not(span_same))
            def _slowspan():
                for u in range(UNROLL):
                    grp16(base + u * 16)

            return c

        return grpu

    def process(bufx, bufb, ngrp):
        first = bufb[pl.ds(0, 16)]
        opensm[0] = first[0]
        lax.fori_loop(0, ngrp // UNROLL, make_grpu(bufx, bufb), 0)
        flush()

    start_blk(xv0, bv0, sem0, wid)
    start_blk(xv1, bv1, sem1, wid + NW)

    def pipe(t, c):
        wait_blk(xv0, bv0, sem0)
        process(xv0, bv0, BLK_ROWS // 16)
        start_blk(xv0, bv0, sem0, wid + NW * (2 * t + 2))
        wait_blk(xv1, bv1, sem1)
        process(xv1, bv1, BLK_ROWS // 16)
        start_blk(xv1, bv1, sem1, wid + NW * (2 * t + 3))
        return c

    lax.fori_loop(0, MAIN_BLKS // 2 - 1, pipe, 0)
    wait_blk(xv0, bv0, sem0)
    process(xv0, bv0, BLK_ROWS // 16)
    wait_blk(xv1, bv1, sem1)
    process(xv1, bv1, BLK_ROWS // 16)

    xoff = NW * MAIN_BLKS * BLK_ROWS

    @pl.when(wid < EXTRA_W - 1)
    def _extra_full():
        off = xoff + wid * BLK_ROWS
        pltpu.sync_copy(x_hbm.at[:, pl.ds(off, BLK_ROWS)], xv0)
        pltpu.sync_copy(b_hbm.at[pl.ds(off, BLK_ROWS)], bv0)
        process(xv0, bv0, BLK_ROWS // 16)

    @pl.when(wid == EXTRA_W - 1)
    def _extra_tail():
        off = xoff + (EXTRA_W - 1) * BLK_ROWS
        pltpu.sync_copy(
            x_hbm.at[:, pl.ds(off, LAST_ROWS)], xv0.at[:, pl.ds(0, LAST_ROWS)]
        )
        pltpu.sync_copy(b_hbm.at[pl.ds(off, LAST_ROWS)], bv0.at[pl.ds(0, LAST_ROWS)])
        process(xv0, bv0, LAST_ROWS // 16)

    pltpu.sync_copy(accx, outx_hbm.at[wid])
    pltpu.sync_copy(accc, outc_hbm.at[wid])


def _tc_phase2_body(ps_ref, pc_ref, prm_ref, pred_ref, lig_ref):
    s = jnp.sum(ps_ref[...], axis=0, keepdims=True)
    cnt = jnp.sum(pc_ref[...], axis=0, keepdims=True)
    cf = jnp.maximum(cnt.astype(jnp.float32), 1.0)
    mx = s[:, 0:S] / cf
    my = s[:, S:2 * S] / cf
    mz = s[:, 2 * S:3 * S] / cf
    mu = (mx + my + mz) * (1.0 / 3.0)
    dx = mx - mu
    dy = my - mu
    dz = mz - mu
    var = (dx * dx + dy * dy + dz * dz) * (1.0 / 3.0)
    rs = lax.rsqrt(var + EPS)
    g0 = prm_ref[0]
    g1 = prm_ref[1]
    g2 = prm_ref[2]
    be0 = prm_ref[3]
    be1 = prm_ref[4]
    be2 = prm_ref[5]
    w0 = prm_ref[6]
    w1 = prm_ref[7]
    w2 = prm_ref[8]
    bb = prm_ref[9]
    x0 = dx * rs * g0 + be0
    x1 = dy * rs * g1 + be1
    x2 = dz * rs * g2 + be2
    pred_ref[...] = x0 * w0 + x1 * w1 + x2 * w2 + bb
    lig_ref[...] = cnt


@jax.jit
def kernel(x_t, batch, ln_gamma, ln_beta, W, b):
    x_cols = x_t.T
    sums_p, cnt_p = _build_phase1()(x_cols, batch)
    params = jnp.concatenate(
        [ln_gamma, ln_beta, W.reshape(-1), b, jnp.zeros((6,), jnp.float32)]
    )
    pred, lig = pl.pallas_call(
        _tc_phase2_body,
        out_shape=[
            jax.ShapeDtypeStruct((1, S), jnp.float32),
            jax.ShapeDtypeStruct((1, S), jnp.int32),
        ],
        in_specs=[
            pl.BlockSpec(memory_space=pltpu.VMEM),
            pl.BlockSpec(memory_space=pltpu.VMEM),
            pl.BlockSpec(memory_space=pltpu.SMEM),
        ],
    )(sums_p, cnt_p, params)
    return pred.reshape(S, 1), lig.reshape(S)
